# baseline (speedup 1.0000x reference)
import functools

import jax
import jax.numpy as jnp
from jax.experimental import pallas as pl
from jax.experimental.pallas import tpu as pltpu

_MOMENTUM = 0.05
_EPS = 1e-06


def _shift_right(z, d, fill):
    pad = jnp.full(z.shape[:-1] + (d,), fill, z.dtype)
    return jnp.concatenate([pad, z[..., :-d]], axis=-1)


def _tsnorm_kernel(x_ref, g_ref, wab_ref, wow_ref, wob_ref, o_ref,
                   carry_ref, *, tb):
    t_idx = pl.program_id(1)

    @pl.when(t_idx == 0)
    def _():
        carry_ref[0] = 0.0
        carry_ref[1] = 0.0

    x = x_ref[0]
    gm = g_ref[0] * _MOMENTUM
    a = 1.0 - gm

    wab = jax.nn.softmax(wab_ref[...], axis=1)

    ra_rb1 = jnp.dot(wab, x, preferred_element_type=jnp.float32)
    rb2 = jnp.dot(wab[1:2, :], x * x,
                  preferred_element_type=jnp.float32)
    ra = ra_rb1[0:1, :]
    rb1 = ra_rb1[1:2, :]

    A = a
    Bm = gm * ra
    a_levels = []
    d = 1
    while d < tb:
        a_levels.append(A)
        Bm = Bm + A * _shift_right(Bm, d, 0.0)
        A = A * _shift_right(A, d, 1.0)
        d *= 2
    m = Bm + A * carry_ref[0]

    Bv = gm * (rb2 - 2.0 * m * rb1 + m * m)
    d = 1
    for Al in a_levels:
        Bv = Bv + Al * _shift_right(Bv, d, 0.0)
        d *= 2
    v = Bv + A * carry_ref[1]

    carry_ref[0] = m[0, tb - 1]
    carry_ref[1] = v[0, tb - 1]

    inv = jax.lax.rsqrt(v + _EPS)
    o_ref[0] = (x - m) * inv * wow_ref[...] + wob_ref[...]


@jax.jit
def kernel(x, g, Wa_w, Wb_w, Wo_w, Wo_b):
    B, C, T = x.shape
    TB = 2048
    grid = (B, T // TB)

    wab = jnp.concatenate([Wa_w.reshape(1, C), Wb_w.reshape(1, C)], axis=0)

    out = pl.pallas_call(
        functools.partial(_tsnorm_kernel, tb=TB),
        out_shape=jax.ShapeDtypeStruct((B, C, T), x.dtype),
        grid=grid,
        in_specs=[
            pl.BlockSpec((1, C, TB), lambda b, t: (b, 0, t)),
            pl.BlockSpec((1, 1, TB), lambda b, t: (b, 0, t)),
            pl.BlockSpec((2, C), lambda b, t: (0, 0)),
            pl.BlockSpec((C, 1), lambda b, t: (0, 0)),
            pl.BlockSpec((C, 1), lambda b, t: (0, 0)),
        ],
        out_specs=pl.BlockSpec((1, C, TB), lambda b, t: (b, 0, t)),
        scratch_shapes=[pltpu.SMEM((2,), jnp.float32)],
        compiler_params=pltpu.CompilerParams(
            dimension_semantics=("parallel", "arbitrary"),
        ),
        name="gated_tsnorm",
    )(
        x,
        g,
        wab,
        Wo_w.reshape(C, 1),
        Wo_b.reshape(C, 1),
    )
    return out


# speedup vs baseline: 1.1833x; 1.1833x over previous
import functools

import jax
import jax.numpy as jnp
from jax.experimental import pallas as pl
from jax.experimental.pallas import tpu as pltpu

_MOMENTUM = 0.05
_EPS = 1e-06


def _shift_right(z, d, fill):
    pad = jnp.full(z.shape[:-1] + (d,), fill, z.dtype)
    return jnp.concatenate([pad, z[..., :-d]], axis=-1)


def _tsnorm_kernel(x_ref, g_ref, wab_ref, wow_ref, wob_ref, o_ref,
                   carry_ref, *, tb):
    t_idx = pl.program_id(1)

    @pl.when(t_idx == 0)
    def _():
        carry_ref[0] = 0.0
        carry_ref[1] = 0.0

    x = x_ref[0]
    gm = g_ref[0] * _MOMENTUM
    a = 1.0 - gm

    wab = jax.nn.softmax(wab_ref[...], axis=1)

    ra_rb1 = jnp.dot(wab, x, preferred_element_type=jnp.float32)
    rb2 = jnp.dot(wab[1:2, :], x * x,
                  preferred_element_type=jnp.float32)
    ra = ra_rb1[0:1, :]
    rb1 = ra_rb1[1:2, :]

    A = a
    Bm = gm * ra
    a_levels = []
    d = 1
    while d < tb:
        a_levels.append(A)
        Bm = Bm + A * _shift_right(Bm, d, 0.0)
        A = A * _shift_right(A, d, 1.0)
        d *= 2
    m = Bm + A * carry_ref[0]

    Bv = gm * (rb2 - 2.0 * m * rb1 + m * m)
    d = 1
    for Al in a_levels:
        Bv = Bv + Al * _shift_right(Bv, d, 0.0)
        d *= 2
    v = Bv + A * carry_ref[1]

    carry_ref[0] = m[0, tb - 1]
    carry_ref[1] = v[0, tb - 1]

    inv = jax.lax.rsqrt(v + _EPS)
    o_ref[0] = (x - m) * inv * wow_ref[...] + wob_ref[...]


@jax.jit
def kernel(x, g, Wa_w, Wb_w, Wo_w, Wo_b):
    B, C, T = x.shape
    TB = 4096
    grid = (B, T // TB)

    wab = jnp.concatenate([Wa_w.reshape(1, C), Wb_w.reshape(1, C)], axis=0)

    out = pl.pallas_call(
        functools.partial(_tsnorm_kernel, tb=TB),
        out_shape=jax.ShapeDtypeStruct((B, C, T), x.dtype),
        grid=grid,
        in_specs=[
            pl.BlockSpec((1, C, TB), lambda b, t: (b, 0, t)),
            pl.BlockSpec((1, 1, TB), lambda b, t: (b, 0, t)),
            pl.BlockSpec((2, C), lambda b, t: (0, 0)),
            pl.BlockSpec((C, 1), lambda b, t: (0, 0)),
            pl.BlockSpec((C, 1), lambda b, t: (0, 0)),
        ],
        out_specs=pl.BlockSpec((1, C, TB), lambda b, t: (b, 0, t)),
        scratch_shapes=[pltpu.SMEM((2,), jnp.float32)],
        compiler_params=pltpu.CompilerParams(
            dimension_semantics=("parallel", "arbitrary"),
        ),
        name="gated_tsnorm",
    )(
        x,
        g,
        wab,
        Wo_w.reshape(C, 1),
        Wo_b.reshape(C, 1),
    )
    return out
